# baseline (speedup 1.0000x reference)
import functools

import numpy as np
import jax
import jax.numpy as jnp
from jax import lax
from jax.experimental import pallas as pl
from jax.experimental.pallas import tpu as pltpu
from jax.experimental.pallas import tpu_sc as plsc

_VOCAB = 1000000
_D = 64
_MAX_LEN = 256
_B, _L = 4096, 200
_TOK = _B * _L
_NW = 32
_PER_W = _TOK // _NW
_CHUNK = 128
_NCH = _PER_W // _CHUNK


def _sinusoidal_pe(max_len, d):
    position = np.arange(max_len, dtype=np.float32)[:, None]
    div_term = np.exp(
        np.arange(0, d, 2, dtype=np.float32) * (-np.log(10000.0) / d))
    pe = np.zeros((max_len, d), dtype=np.float32)
    pe[:, 0::2] = np.sin(position * div_term)
    pe[:, 1::2] = np.cos(position * div_term)
    return pe


_PE = jnp.asarray(_sinusoidal_pe(_MAX_LEN, _D)[:_L])

_BCAST_DNUMS = lax.GatherDimensionNumbers(
    offset_dims=(), collapsed_slice_dims=(0,), start_index_map=(0,))


def _bcast_lane(vec16, r):
    idx = jnp.full((16, 1), r, jnp.int32)
    return lax.gather(vec16, idx, _BCAST_DNUMS, slice_sizes=(1,),
                      mode=lax.GatherScatterMode.PROMISE_IN_BOUNDS)


def _emb_body(tokens_hbm, table_hbm, pe_hbm, out_hbm,
              idx_v, rows_v, pe_v, sem):
    w = lax.axis_index("s") * 2 + lax.axis_index("c")
    base = w * _PER_W

    pltpu.sync_copy(pe_hbm, pe_v)

    def chunk_body(c, carry):
        flat0 = base + c * _CHUNK
        pltpu.sync_copy(tokens_hbm.at[pl.ds(flat0, _CHUNK)], idx_v)
        pltpu.async_copy(table_hbm.at[idx_v], rows_v, sem).wait()

        base_l = lax.rem(c * _CHUNK, _L)

        def grp_body(g, gcarry):
            tok16 = idx_v[pl.ds(g * 16, 16)]
            m16 = jnp.where(tok16 == 0, 0.0, 1.0).astype(jnp.float32)

            def row_body(r, rcarry):
                j = g * 16 + r
                l = lax.rem(base_l + j, _L)
                mb = _bcast_lane(m16, r)
                for d in range(_D // 16):
                    sl = pl.ds(d * 16, 16)
                    rows_v[j, sl] = rows_v[j, sl] * mb + pe_v[l, sl]
                return rcarry

            return lax.fori_loop(0, 16, row_body, gcarry)

        lax.fori_loop(0, _CHUNK // 16, grp_body, 0)

        pltpu.sync_copy(rows_v, out_hbm.at[pl.ds(flat0, _CHUNK), :])
        return carry

    lax.fori_loop(0, _NCH, chunk_body, 0)


@jax.jit
def _emb(tokens_flat, table, pe):
    mesh = plsc.VectorSubcoreMesh(core_axis_name="c", subcore_axis_name="s")
    run = functools.partial(
        pl.kernel,
        out_type=jax.ShapeDtypeStruct((_TOK, _D), jnp.float32),
        mesh=mesh,
        scratch_types=[
            pltpu.VMEM((_CHUNK,), jnp.int32),
            pltpu.VMEM((_CHUNK, _D), jnp.float32),
            pltpu.VMEM((_L, _D), jnp.float32),
            pltpu.SemaphoreType.DMA,
        ],
        compiler_params=pltpu.CompilerParams(use_tc_tiling_on_sc=False),
    )(_emb_body)
    return run(tokens_flat, table, pe)


def kernel(tokens, table):
    tokens_flat = tokens.reshape(-1).astype(jnp.int32)
    out = _emb(tokens_flat, table, _PE)
    return out.reshape(_B, _L, _D)


# speedup vs baseline: 1.1165x; 1.1165x over previous
import functools

import numpy as np
import jax
import jax.numpy as jnp
from jax import lax
from jax.experimental import pallas as pl
from jax.experimental.pallas import tpu as pltpu
from jax.experimental.pallas import tpu_sc as plsc

_VOCAB = 1000000
_D = 64
_MAX_LEN = 256
_B, _L = 4096, 200
_TOK = _B * _L
_NW = 32
_PER_W = _TOK // _NW
_CHUNK = 128
_NCH = _PER_W // _CHUNK


def _sinusoidal_pe(max_len, d):
    position = np.arange(max_len, dtype=np.float32)[:, None]
    div_term = np.exp(
        np.arange(0, d, 2, dtype=np.float32) * (-np.log(10000.0) / d))
    pe = np.zeros((max_len, d), dtype=np.float32)
    pe[:, 0::2] = np.sin(position * div_term)
    pe[:, 1::2] = np.cos(position * div_term)
    return pe


_PE = jnp.asarray(_sinusoidal_pe(_MAX_LEN, _D)[:_L])

_BCAST_DNUMS = lax.GatherDimensionNumbers(
    offset_dims=(), collapsed_slice_dims=(0,), start_index_map=(0,))


def _bcast_lane(vec16, r):
    idx = jnp.full((16, 1), r, jnp.int32)
    return lax.gather(vec16, idx, _BCAST_DNUMS, slice_sizes=(1,),
                      mode=lax.GatherScatterMode.PROMISE_IN_BOUNDS)


_NBUF = 4
_NSTEP = _NCH // _NBUF


def _emb_body(tokens_hbm, table_hbm, pe_hbm, out_hbm,
              idx_v, rows_v, obuf_v, pe_v, gsem, osem):
    w = lax.axis_index("s") * 2 + lax.axis_index("c")
    base = w * _PER_W

    pltpu.sync_copy(tokens_hbm.at[pl.ds(base, _PER_W)], idx_v)
    pltpu.sync_copy(pe_hbm, pe_v)

    def start_gather(c, b):
        pltpu.async_copy(
            table_hbm.at[idx_v.at[pl.ds(c * _CHUNK, _CHUNK)]],
            rows_v.at[b], gsem.at[b])

    for b in range(_NBUF):
        start_gather(b, b)

    def step_body(step, carry):
        for b in range(_NBUF):
            c = step * _NBUF + b
            off = c * _CHUNK
            base_l = lax.rem(off, _L)

            pltpu.make_async_copy(
                table_hbm.at[idx_v.at[pl.ds(off, _CHUNK)]],
                rows_v.at[b], gsem.at[b]).wait()

            @pl.when(step > 0)
            def _drain():
                pltpu.make_async_copy(
                    obuf_v.at[b],
                    out_hbm.at[pl.ds(base + (c - _NBUF) * _CHUNK, _CHUNK), :],
                    osem.at[b]).wait()

            def grp_body(g, gcarry):
                tok16 = idx_v[pl.ds(off + g * 16, 16)]
                m16 = jnp.where(tok16 == 0, 0.0, 1.0).astype(jnp.float32)

                def row_body(r, rcarry):
                    j = g * 16 + r
                    l = lax.rem(base_l + j, _L)
                    mb = _bcast_lane(m16, r)
                    for d in range(_D // 16):
                        sl = pl.ds(d * 16, 16)
                        obuf_v[b, j, sl] = rows_v[b, j, sl] * mb + pe_v[l, sl]
                    return rcarry

                return lax.fori_loop(0, 16, row_body, gcarry)

            lax.fori_loop(0, _CHUNK // 16, grp_body, 0)

            @pl.when(step < _NSTEP - 1)
            def _refill():
                start_gather(c + _NBUF, b)

            pltpu.async_copy(
                obuf_v.at[b],
                out_hbm.at[pl.ds(base + off, _CHUNK), :],
                osem.at[b])
        return carry

    lax.fori_loop(0, _NSTEP, step_body, 0)

    for b in range(_NBUF):
        c = (_NSTEP - 1) * _NBUF + b
        pltpu.make_async_copy(
            obuf_v.at[b],
            out_hbm.at[pl.ds(base + c * _CHUNK, _CHUNK), :],
            osem.at[b]).wait()


@jax.jit
def _emb(tokens_flat, table, pe):
    mesh = plsc.VectorSubcoreMesh(core_axis_name="c", subcore_axis_name="s")
    run = functools.partial(
        pl.kernel,
        out_type=jax.ShapeDtypeStruct((_TOK, _D), jnp.float32),
        mesh=mesh,
        scratch_types=[
            pltpu.VMEM((_PER_W,), jnp.int32),
            pltpu.VMEM((_NBUF, _CHUNK, _D), jnp.float32),
            pltpu.VMEM((_NBUF, _CHUNK, _D), jnp.float32),
            pltpu.VMEM((_L, _D), jnp.float32),
            pltpu.SemaphoreType.DMA((_NBUF,)),
            pltpu.SemaphoreType.DMA((_NBUF,)),
        ],
        compiler_params=pltpu.CompilerParams(use_tc_tiling_on_sc=False),
    )(_emb_body)
    return run(tokens_flat, table, pe)


def kernel(tokens, table):
    tokens_flat = tokens.reshape(-1).astype(jnp.int32)
    out = _emb(tokens_flat, table, _PE)
    return out.reshape(_B, _L, _D)


# speedup vs baseline: 1.4763x; 1.3222x over previous
import functools

import numpy as np
import jax
import jax.numpy as jnp
from jax import lax
from jax.experimental import pallas as pl
from jax.experimental.pallas import tpu as pltpu
from jax.experimental.pallas import tpu_sc as plsc

_VOCAB = 1000000
_D = 64
_MAX_LEN = 256
_B, _L = 4096, 200
_NW = 32
_SEQ_PER_W = _B // _NW
_GCH = 40
_NGCH = _L // _GCH
_NBUF = 4
_NSTEP = _SEQ_PER_W // _NBUF


def _sinusoidal_pe(max_len, d):
    position = np.arange(max_len, dtype=np.float32)[:, None]
    div_term = np.exp(
        np.arange(0, d, 2, dtype=np.float32) * (-np.log(10000.0) / d))
    pe = np.zeros((max_len, d), dtype=np.float32)
    pe[:, 0::2] = np.sin(position * div_term)
    pe[:, 1::2] = np.cos(position * div_term)
    return pe


_PE = jnp.asarray(_sinusoidal_pe(_MAX_LEN, _D)[:_L])

_BCAST_DNUMS = lax.GatherDimensionNumbers(
    offset_dims=(), collapsed_slice_dims=(0,), start_index_map=(0,))


def _bcast_lane(vec16, r):
    idx = jnp.full((16, 1), r, jnp.int32)
    return lax.gather(vec16, idx, _BCAST_DNUMS, slice_sizes=(1,),
                      mode=lax.GatherScatterMode.PROMISE_IN_BOUNDS)


def _emb_body(tokens_hbm, table_hbm, pe_hbm, out_hbm,
              idx_v, rows_v, pe_v, gsem, osem):
    w = lax.axis_index("s") * 2 + lax.axis_index("c")
    seq0 = w * _SEQ_PER_W

    pltpu.sync_copy(tokens_hbm.at[pl.ds(seq0, _SEQ_PER_W), :], idx_v)
    pltpu.sync_copy(pe_hbm, pe_v)

    def start_gather(i, b):
        for k in range(_NGCH):
            pltpu.async_copy(
                table_hbm.at[idx_v.at[i, pl.ds(k * _GCH, _GCH)]],
                rows_v.at[b, pl.ds(k * _GCH, _GCH), :], gsem.at[b])

    def wait_gather(b):
        pltpu.make_async_copy(
            table_hbm.at[pl.ds(0, _L), :], rows_v.at[b], gsem.at[b]).wait()

    def wait_out(b):
        pltpu.make_async_copy(
            rows_v.at[b], out_hbm.at[0], osem.at[b]).wait()

    def compute(i, b):
        def grp_body(g, gcarry):
            tok16 = idx_v[i, pl.ds(g * 16, 16)]
            m16 = jnp.where(tok16 == 0, 0.0, 1.0).astype(jnp.float32)
            for r in range(16):
                mb = _bcast_lane(m16, r)
                j = g * 16 + r
                for d in range(_D // 16):
                    sl = pl.ds(d * 16, 16)
                    rows_v[b, j, sl] = rows_v[b, j, sl] * mb + pe_v[j, sl]
            return gcarry

        lax.fori_loop(0, _L // 16, grp_body, 0)
        tok16 = idx_v[i, pl.ds(_L - 16, 16)]
        m16 = jnp.where(tok16 == 0, 0.0, 1.0).astype(jnp.float32)
        for r in range(8):
            mb = _bcast_lane(m16, 8 + r)
            j = _L - 8 + r
            for d in range(_D // 16):
                sl = pl.ds(d * 16, 16)
                rows_v[b, j, sl] = rows_v[b, j, sl] * mb + pe_v[j, sl]

    start_gather(0, 0)
    start_gather(1, 1)

    def step_body(step, carry):
        for b in range(_NBUF):
            i = step * _NBUF + b
            b2 = (b + 2) % _NBUF

            if b < 2:
                @pl.when(step > 0)
                def _recycle():
                    wait_out(b2)
            else:
                wait_out(b2)
            if b < 2:
                start_gather(i + 2, b2)
            else:
                @pl.when(step < _NSTEP - 1)
                def _ahead():
                    start_gather(i + 2, b2)

            wait_gather(b)
            compute(i, b)
            pltpu.async_copy(rows_v.at[b], out_hbm.at[seq0 + i], osem.at[b])
        return carry

    lax.fori_loop(0, _NSTEP, step_body, 0)

    wait_out(2)
    wait_out(3)


@jax.jit
def _emb(tokens, table, pe):
    mesh = plsc.VectorSubcoreMesh(core_axis_name="c", subcore_axis_name="s")
    run = functools.partial(
        pl.kernel,
        out_type=jax.ShapeDtypeStruct((_B, _L, _D), jnp.float32),
        mesh=mesh,
        scratch_types=[
            pltpu.VMEM((_SEQ_PER_W, _L), jnp.int32),
            pltpu.VMEM((_NBUF, _L, _D), jnp.float32),
            pltpu.VMEM((_L, _D), jnp.float32),
            pltpu.SemaphoreType.DMA((_NBUF,)),
            pltpu.SemaphoreType.DMA((_NBUF,)),
        ],
        compiler_params=pltpu.CompilerParams(use_tc_tiling_on_sc=False),
    )(_emb_body)
    return run(tokens, table, pe)


def kernel(tokens, table):
    return _emb(tokens.astype(jnp.int32), table, _PE)


# speedup vs baseline: 1.5931x; 1.0791x over previous
import functools

import numpy as np
import jax
import jax.numpy as jnp
from jax import lax
from jax.experimental import pallas as pl
from jax.experimental.pallas import tpu as pltpu
from jax.experimental.pallas import tpu_sc as plsc

_VOCAB = 1000000
_D = 64
_MAX_LEN = 256
_B, _L = 4096, 200
_NW = 32
_BW = _B // _NW
_NBUF = 4
_NSTEP = _L // _NBUF


def _sinusoidal_pe(max_len, d):
    position = np.arange(max_len, dtype=np.float32)[:, None]
    div_term = np.exp(
        np.arange(0, d, 2, dtype=np.float32) * (-np.log(10000.0) / d))
    pe = np.zeros((max_len, d), dtype=np.float32)
    pe[:, 0::2] = np.sin(position * div_term)
    pe[:, 1::2] = np.cos(position * div_term)
    return pe


_PE = jnp.asarray(_sinusoidal_pe(_MAX_LEN, _D)[:_L])

_BCAST_DNUMS = lax.GatherDimensionNumbers(
    offset_dims=(), collapsed_slice_dims=(0,), start_index_map=(0,))


def _bcast_lane(vec16, r):
    idx = jnp.full((16, 1), r, jnp.int32)
    return lax.gather(vec16, idx, _BCAST_DNUMS, slice_sizes=(1,),
                      mode=lax.GatherScatterMode.PROMISE_IN_BOUNDS)


def _emb_body(tokens_t_hbm, table_hbm, pe_hbm, out_hbm,
              idx_v, rows_v, pe_v, gsem, osem):
    w = lax.axis_index("s") * 2 + lax.axis_index("c")
    b0 = w * _BW

    pltpu.sync_copy(tokens_t_hbm.at[:, pl.ds(b0, _BW)], idx_v)
    pltpu.sync_copy(pe_hbm, pe_v)

    def start_gather(l, b):
        pltpu.async_copy(
            table_hbm.at[idx_v.at[l, :]], rows_v.at[b], gsem.at[b])

    def wait_gather(b):
        pltpu.make_async_copy(
            table_hbm.at[pl.ds(0, _BW), :], rows_v.at[b], gsem.at[b]).wait()

    def wait_out(b):
        pltpu.make_async_copy(
            rows_v.at[b], out_hbm.at[0, pl.ds(0, _BW), :], osem.at[b]).wait()

    def compute(l, b):
        pe_r = [pe_v[l, pl.ds(d * 16, 16)] for d in range(_D // 16)]

        def grp_body(g, gcarry):
            tok16 = idx_v[l, pl.ds(g * 16, 16)]
            m16 = jnp.where(tok16 == 0, 0.0, 1.0).astype(jnp.float32)
            for r in range(16):
                mb = _bcast_lane(m16, r)
                j = g * 16 + r
                for d in range(_D // 16):
                    sl = pl.ds(d * 16, 16)
                    rows_v[b, j, sl] = rows_v[b, j, sl] * mb + pe_r[d]
            return gcarry

        lax.fori_loop(0, _BW // 16, grp_body, 0)

    start_gather(0, 0)
    start_gather(1, 1)

    def step_body(step, carry):
        for b in range(_NBUF):
            l = step * _NBUF + b
            b2 = (b + 2) % _NBUF

            if b < 2:
                @pl.when(step > 0)
                def _recycle():
                    wait_out(b2)
            else:
                wait_out(b2)
            if b < 2:
                start_gather(l + 2, b2)
            else:
                @pl.when(step < _NSTEP - 1)
                def _ahead():
                    start_gather(l + 2, b2)

            wait_gather(b)
            compute(l, b)
            pltpu.async_copy(
                rows_v.at[b], out_hbm.at[l, pl.ds(b0, _BW), :], osem.at[b])
        return carry

    lax.fori_loop(0, _NSTEP, step_body, 0)

    wait_out(2)
    wait_out(3)


@jax.jit
def _emb(tokens_t, table, pe):
    mesh = plsc.VectorSubcoreMesh(core_axis_name="c", subcore_axis_name="s")
    run = functools.partial(
        pl.kernel,
        out_type=jax.ShapeDtypeStruct((_L, _B, _D), jnp.float32),
        mesh=mesh,
        scratch_types=[
            pltpu.VMEM((_L, _BW), jnp.int32),
            pltpu.VMEM((_NBUF, _BW, _D), jnp.float32),
            pltpu.VMEM((_L, _D), jnp.float32),
            pltpu.SemaphoreType.DMA((_NBUF,)),
            pltpu.SemaphoreType.DMA((_NBUF,)),
        ],
        compiler_params=pltpu.CompilerParams(use_tc_tiling_on_sc=False),
    )(_emb_body)
    return run(tokens_t, table, pe)


def kernel(tokens, table):
    out_t = _emb(tokens.astype(jnp.int32).T, table, _PE)
    return out_t.transpose(1, 0, 2)
